# initial kernel scaffold (unmeasured)
import jax
import jax.numpy as jnp
from jax import lax
from jax.experimental import pallas as pl
from jax.experimental.pallas import tpu as pltpu

N_DEV = 4
R = 4
BLK = 64
HQ = 8
DH = 128
SCALE = 0.08838834764831843


def kernel(x, Wq, K_ext, V_ext, Wo):
    _, Sq, D = x.shape
    Skv_loc = K_ext.shape[1]
    ngroups = Sq // BLK // R
    rows_r = Sq // R

    def regroup(a):
        c = a.shape[-1]
        return a.reshape(ngroups, R, BLK, c).transpose(1, 0, 2, 3).reshape(-1, c)

    xr = regroup(x[0])
    kr = regroup(K_ext[0].reshape(Skv_loc, HQ * DH))
    vr = regroup(V_ext[0].reshape(Skv_loc, HQ * DH))

    def body(xr_ref, wq_ref, kr_ref, vr_ref, wo_ref, out_ref,
             q_s, ctx_comm, stats_comm,
             send_sems, recv_sems, s_send_sems, s_recv_sems):
        my = lax.axis_index("i")
        right = (my + 1) % N_DEV
        left = (my + N_DEV - 1) % N_DEV

        barrier_sem = pltpu.get_barrier_semaphore()
        for nbr in (left, right):
            pl.semaphore_signal(
                barrier_sem, inc=1,
                device_id=(nbr,), device_id_type=pl.DeviceIdType.MESH,
            )
        pl.semaphore_wait(barrier_sem, 2)

        q_s[:, :] = jnp.dot(xr_ref[:, :], wq_ref[:, :],
                            preferred_element_type=jnp.float32)

        for r in range(R):
            rows = pl.ds(r * rows_r, rows_r)
            ms, ls = [], []
            for h in range(HQ):
                cols = pl.ds(h * DH, DH)
                qh = q_s[rows, cols]
                kh = kr_ref[rows, cols]
                s = lax.dot_general(
                    qh, kh, (((1,), (1,)), ((), ())),
                    preferred_element_type=jnp.float32,
                ) * SCALE
                m = jnp.max(s, axis=1, keepdims=True)
                w = jnp.exp(s - m)
                l = jnp.sum(w, axis=1, keepdims=True)
                ctx_comm[0, rows, cols] = jnp.dot(
                    w, vr_ref[rows, cols],
                    preferred_element_type=jnp.float32)
                ms.append(m)
                ls.append(l)
            stats_comm[0, rows, :] = jnp.concatenate(ms + ls, axis=1)

        for hop in range(N_DEV - 1):
            rc = pltpu.make_async_remote_copy(
                src_ref=ctx_comm.at[hop],
                dst_ref=ctx_comm.at[hop + 1],
                send_sem=send_sems.at[hop],
                recv_sem=recv_sems.at[hop],
                device_id=(right,),
                device_id_type=pl.DeviceIdType.MESH,
            )
            rs = pltpu.make_async_remote_copy(
                src_ref=stats_comm.at[hop],
                dst_ref=stats_comm.at[hop + 1],
                send_sem=s_send_sems.at[hop],
                recv_sem=s_recv_sems.at[hop],
                device_id=(right,),
                device_id_type=pl.DeviceIdType.MESH,
            )
            rc.start()
            rs.start()
            rc.wait()
            rs.wait()

        sts = [stats_comm[s, :, :] for s in range(N_DEV)]
        ms = [st[:, 0:HQ] for st in sts]
        ls = [st[:, HQ:2 * HQ] for st in sts]
        M = jnp.maximum(jnp.maximum(ms[0], ms[1]), jnp.maximum(ms[2], ms[3]))
        coefs = [jnp.exp(mm - M) for mm in ms]
        L = (coefs[0] * ls[0] + coefs[1] * ls[1]
             + coefs[2] * ls[2] + coefs[3] * ls[3])
        fac = [c / L for c in coefs]
        for h in range(HQ):
            cols = pl.ds(h * DH, DH)
            q_s[:, cols] = (ctx_comm[0, :, cols] * fac[0][:, h:h + 1]
                            + ctx_comm[1, :, cols] * fac[1][:, h:h + 1]
                            + ctx_comm[2, :, cols] * fac[2][:, h:h + 1]
                            + ctx_comm[3, :, cols] * fac[3][:, h:h + 1])

        out_ref[:, :] = jnp.dot(q_s[:, :], wo_ref[:, :],
                                preferred_element_type=jnp.float32)

    out2 = pl.pallas_call(
        body,
        out_shape=jax.ShapeDtypeStruct((Sq, HQ * DH), jnp.float32),
        in_specs=[pl.BlockSpec(memory_space=pltpu.VMEM)] * 5,
        out_specs=pl.BlockSpec(memory_space=pltpu.VMEM),
        scratch_shapes=[
            pltpu.VMEM((Sq, HQ * DH), jnp.float32),
            pltpu.VMEM((N_DEV, Sq, HQ * DH), jnp.float32),
            pltpu.VMEM((N_DEV, Sq, 2 * HQ), jnp.float32),
            pltpu.SemaphoreType.DMA((N_DEV - 1,)),
            pltpu.SemaphoreType.DMA((N_DEV - 1,)),
            pltpu.SemaphoreType.DMA((N_DEV - 1,)),
            pltpu.SemaphoreType.DMA((N_DEV - 1,)),
        ],
        compiler_params=pltpu.CompilerParams(collective_id=0),
    )(xr, Wq, kr, vr, Wo)

    out = out2.reshape(R, ngroups, BLK, HQ * DH).transpose(1, 0, 2, 3)
    return out.reshape(1, Sq, HQ * DH)


# baseline (device time: 256592 ns/iter reference)
import jax
import jax.numpy as jnp
from jax import lax
from jax.experimental import pallas as pl
from jax.experimental.pallas import tpu as pltpu

N_DEV = 4
R = 4
BLK = 64
HQ = 8
DH = 128
SCALE = 0.08838834764831843


def kernel(x, Wq, K_ext, V_ext, Wo):
    _, Sq, D = x.shape
    Skv_loc = K_ext.shape[1]
    ngroups = Sq // BLK // R
    rows_r = Sq // R

    def regroup(a):
        c = a.shape[-1]
        return a.reshape(ngroups, R, BLK, c).transpose(1, 0, 2, 3).reshape(-1, c)

    qr = jnp.dot(regroup(x[0]), Wq,
                 preferred_element_type=jnp.float32).astype(jnp.bfloat16)
    kr = regroup(K_ext[0].reshape(Skv_loc, HQ * DH)).astype(jnp.bfloat16)
    vr = regroup(V_ext[0].reshape(Skv_loc, HQ * DH)).astype(jnp.bfloat16)

    def body(q_ref, kr_ref, vr_ref, out_ref,
             ctx_comm, stats_comm,
             send_sems, recv_sems, s_send_sems, s_recv_sems):
        my = lax.axis_index("i")
        right = (my + 1) % N_DEV
        left = (my + N_DEV - 1) % N_DEV

        barrier_sem = pltpu.get_barrier_semaphore()
        for nbr in (left, right):
            pl.semaphore_signal(
                barrier_sem, inc=1,
                device_id=(nbr,), device_id_type=pl.DeviceIdType.MESH,
            )
        pl.semaphore_wait(barrier_sem, 2)

        for h in range(HQ):
            cols = pl.ds(h * DH, DH)

            def rbody(r, _, cols=cols, h=h):
                rows = pl.ds(r * rows_r, rows_r)
                qh = q_ref[rows, cols]
                kh = kr_ref[rows, cols]
                s = lax.dot_general(
                    qh, kh, (((1,), (1,)), ((), ())),
                    preferred_element_type=jnp.float32,
                ) * SCALE
                m = jnp.max(s, axis=1, keepdims=True)
                w = jnp.exp(s - m)
                l = jnp.sum(w, axis=1, keepdims=True)
                ctx = jnp.dot(w.astype(jnp.bfloat16), vr_ref[rows, cols],
                              preferred_element_type=jnp.float32)
                ctx_comm[0, rows, cols] = ctx.astype(jnp.bfloat16)
                out_ref[rows, cols] = ctx
                stats_comm[0, rows, pl.ds(h, 1)] = m
                stats_comm[0, rows, pl.ds(HQ + h, 1)] = l
                return 0

            lax.fori_loop(0, R, rbody, 0)

        m_acc = stats_comm[0, :, 0:HQ]
        l_acc = stats_comm[0, :, HQ:2 * HQ]

        for hop in range(N_DEV - 1):
            s_slot = hop % 2
            r_slot = (hop + 1) % 2
            rc = pltpu.make_async_remote_copy(
                src_ref=ctx_comm.at[s_slot],
                dst_ref=ctx_comm.at[r_slot],
                send_sem=send_sems.at[hop],
                recv_sem=recv_sems.at[hop],
                device_id=(right,),
                device_id_type=pl.DeviceIdType.MESH,
            )
            rs = pltpu.make_async_remote_copy(
                src_ref=stats_comm.at[s_slot],
                dst_ref=stats_comm.at[r_slot],
                send_sem=s_send_sems.at[hop],
                recv_sem=s_recv_sems.at[hop],
                device_id=(right,),
                device_id_type=pl.DeviceIdType.MESH,
            )
            rc.start()
            rs.start()
            rc.wait()
            rs.wait()

            m_in = stats_comm[r_slot, :, 0:HQ]
            l_in = stats_comm[r_slot, :, HQ:2 * HQ]
            m_new = jnp.maximum(m_acc, m_in)
            c_old = jnp.exp(m_acc - m_new)
            c_in = jnp.exp(m_in - m_new)
            l_acc = l_acc * c_old + l_in * c_in
            m_acc = m_new
            if hop == N_DEV - 2:
                c_old = c_old / l_acc
                c_in = c_in / l_acc
            for h in range(HQ):
                cols = pl.ds(h * DH, DH)
                out_ref[:, cols] = (
                    out_ref[:, cols] * c_old[:, h:h + 1]
                    + ctx_comm[r_slot, :, cols].astype(jnp.float32)
                    * c_in[:, h:h + 1])

    ctx2 = pl.pallas_call(
        body,
        out_shape=jax.ShapeDtypeStruct((Sq, HQ * DH), jnp.float32),
        in_specs=[pl.BlockSpec(memory_space=pltpu.VMEM)] * 3,
        out_specs=pl.BlockSpec(memory_space=pltpu.VMEM),
        scratch_shapes=[
            pltpu.VMEM((2, Sq, HQ * DH), jnp.bfloat16),
            pltpu.VMEM((2, Sq, 2 * HQ), jnp.float32),
            pltpu.SemaphoreType.DMA((N_DEV - 1,)),
            pltpu.SemaphoreType.DMA((N_DEV - 1,)),
            pltpu.SemaphoreType.DMA((N_DEV - 1,)),
            pltpu.SemaphoreType.DMA((N_DEV - 1,)),
        ],
        compiler_params=pltpu.CompilerParams(
            collective_id=0, vmem_limit_bytes=44 * 1024 * 1024),
    )(qr, kr, vr)

    ctx_out = ctx2.reshape(R, ngroups, BLK, HQ * DH).transpose(1, 0, 2, 3)
    out = jnp.dot(ctx_out.reshape(Sq, HQ * DH), Wo,
                  preferred_element_type=jnp.float32)
    return out[None]


# device time: 172183 ns/iter; 1.4902x vs baseline; 1.4902x over previous
import jax
import jax.numpy as jnp
from jax import lax
from jax.experimental import pallas as pl
from jax.experimental.pallas import tpu as pltpu

N_DEV = 4
R = 4
BLK = 64
HQ = 8
DH = 128
SCALE = 0.08838834764831843


def kernel(x, Wq, K_ext, V_ext, Wo):
    _, Sq, D = x.shape
    Skv_loc = K_ext.shape[1]
    ngroups = Sq // BLK // R
    rows_r = Sq // R

    def regroup(a):
        c = a.shape[-1]
        return a.reshape(ngroups, R, BLK, c).transpose(1, 0, 2, 3).reshape(-1, c)

    qr = jnp.dot(regroup(x[0]), Wq,
                 preferred_element_type=jnp.float32).astype(jnp.bfloat16)
    kr = regroup(K_ext[0].reshape(Skv_loc, HQ * DH)).astype(jnp.bfloat16)
    vr = regroup(V_ext[0].reshape(Skv_loc, HQ * DH)).astype(jnp.bfloat16)

    def body(q_ref, kr_ref, vr_ref, out_ref,
             ctx_comm, stats_comm, send_sems, recv_sems):
        my = lax.axis_index("i")
        right = (my + 1) % N_DEV
        left = (my + N_DEV - 1) % N_DEV

        barrier_sem = pltpu.get_barrier_semaphore()
        for nbr in (left, right):
            pl.semaphore_signal(
                barrier_sem, inc=1,
                device_id=(nbr,), device_id_type=pl.DeviceIdType.MESH,
            )
        pl.semaphore_wait(barrier_sem, 2)

        for h in range(HQ):
            cols = pl.ds(h * DH, DH)

            def rbody(r, _, cols=cols, h=h):
                rows = pl.ds(r * rows_r, rows_r)
                qh = q_ref[rows, cols]
                kh = kr_ref[rows, cols]
                s = lax.dot_general(
                    qh, kh, (((1,), (1,)), ((), ())),
                    preferred_element_type=jnp.float32,
                ) * SCALE
                m = jnp.max(s, axis=1, keepdims=True)
                w = jnp.exp(s - m)
                l = jnp.sum(w, axis=1, keepdims=True)
                ctx = jnp.dot(w.astype(jnp.bfloat16), vr_ref[rows, cols],
                              preferred_element_type=jnp.float32)
                ctx_comm[0, rows, cols] = ctx.astype(jnp.bfloat16)
                out_ref[rows, cols] = ctx
                stats_comm[0, rows, pl.ds(h, 1)] = m
                stats_comm[0, rows, pl.ds(HQ + h, 1)] = l
                return 0

            lax.fori_loop(0, R, rbody, 0)

        m_acc = stats_comm[0, :, 0:HQ]
        l_acc = stats_comm[0, :, HQ:2 * HQ]

        half = Sq // 2
        for hop in range(N_DEV - 1):
            s_slot = hop % 2
            r_slot = (hop + 1) % 2
            flows = []
            for f, (buf, nbr, rows) in enumerate((
                    (ctx_comm, right, pl.ds(0, half)),
                    (ctx_comm, left, pl.ds(half, half)),
                    (stats_comm, right, pl.ds(0, half)),
                    (stats_comm, left, pl.ds(half, half)))):
                flows.append(pltpu.make_async_remote_copy(
                    src_ref=buf.at[s_slot, rows],
                    dst_ref=buf.at[r_slot, rows],
                    send_sem=send_sems.at[hop, f],
                    recv_sem=recv_sems.at[hop, f],
                    device_id=(nbr,),
                    device_id_type=pl.DeviceIdType.MESH,
                ))
            for fl in flows:
                fl.start()
            for fl in flows:
                fl.wait()

            m_in = stats_comm[r_slot, :, 0:HQ]
            l_in = stats_comm[r_slot, :, HQ:2 * HQ]
            m_new = jnp.maximum(m_acc, m_in)
            c_old = jnp.exp(m_acc - m_new)
            c_in = jnp.exp(m_in - m_new)
            l_acc = l_acc * c_old + l_in * c_in
            m_acc = m_new
            if hop == N_DEV - 2:
                c_old = c_old / l_acc
                c_in = c_in / l_acc
            for h in range(HQ):
                cols = pl.ds(h * DH, DH)
                out_ref[:, cols] = (
                    out_ref[:, cols] * c_old[:, h:h + 1]
                    + ctx_comm[r_slot, :, cols].astype(jnp.float32)
                    * c_in[:, h:h + 1])

    ctx2 = pl.pallas_call(
        body,
        out_shape=jax.ShapeDtypeStruct((Sq, HQ * DH), jnp.float32),
        in_specs=[pl.BlockSpec(memory_space=pltpu.VMEM)] * 3,
        out_specs=pl.BlockSpec(memory_space=pltpu.VMEM),
        scratch_shapes=[
            pltpu.VMEM((2, Sq, HQ * DH), jnp.bfloat16),
            pltpu.VMEM((2, Sq, 2 * HQ), jnp.float32),
            pltpu.SemaphoreType.DMA((N_DEV - 1, 4)),
            pltpu.SemaphoreType.DMA((N_DEV - 1, 4)),
        ],
        compiler_params=pltpu.CompilerParams(
            collective_id=0, vmem_limit_bytes=44 * 1024 * 1024),
    )(qr, kr, vr)

    ctx_out = ctx2.reshape(R, ngroups, BLK, HQ * DH).transpose(1, 0, 2, 3)
    out = jnp.dot(ctx_out.reshape(Sq, HQ * DH), Wo,
                  preferred_element_type=jnp.float32)
    return out[None]


# device time: 125334 ns/iter; 2.0473x vs baseline; 1.3738x over previous
import jax
import jax.numpy as jnp
from jax import lax
from jax.experimental import pallas as pl
from jax.experimental.pallas import tpu as pltpu

N_DEV = 4
R = 4
BLK = 64
HQ = 8
DH = 128
SCALE = 0.08838834764831843


def kernel(x, Wq, K_ext, V_ext, Wo):
    _, Sq, D = x.shape
    Skv_loc = K_ext.shape[1]
    ngroups = Sq // BLK // R
    rows_r = Sq // R

    def regroup(a):
        c = a.shape[-1]
        return a.reshape(ngroups, R, BLK, c).transpose(1, 0, 2, 3).reshape(-1, c)

    qr = (jnp.dot(regroup(x[0]).astype(jnp.bfloat16), Wq.astype(jnp.bfloat16),
                  preferred_element_type=jnp.float32)
          * SCALE).astype(jnp.bfloat16)
    kr = regroup(K_ext[0].reshape(Skv_loc, HQ * DH)).astype(jnp.bfloat16)
    vr = regroup(V_ext[0].reshape(Skv_loc, HQ * DH)).astype(jnp.bfloat16)

    def body(q_ref, kr_ref, vr_ref, out_ref,
             ctx_comm, stats_comm, local_stats, send_sems, recv_sems):
        my = lax.axis_index("i")
        right = (my + 1) % N_DEV
        left = (my + N_DEV - 1) % N_DEV

        barrier_sem = pltpu.get_barrier_semaphore()
        for nbr in (left, right):
            pl.semaphore_signal(
                barrier_sem, inc=1,
                device_id=(nbr,), device_id_type=pl.DeviceIdType.MESH,
            )
        pl.semaphore_wait(barrier_sem, 2)

        def attn_head(h):
            cols = pl.ds(h * DH, DH)
            m_lane = (h // 2) * 4 + h % 2

            def rbody(r, _, cols=cols, m_lane=m_lane):
                rows = pl.ds(r * rows_r, rows_r)
                qh = q_ref[rows, cols]
                kh = kr_ref[rows, cols]
                s = lax.dot_general(
                    qh, kh, (((1,), (1,)), ((), ())),
                    preferred_element_type=jnp.float32,
                )
                m = jnp.max(s, axis=1, keepdims=True)
                w = jnp.exp(s - m)
                l = jnp.sum(w, axis=1, keepdims=True)
                ctx = jnp.dot(w.astype(jnp.bfloat16), vr_ref[rows, cols],
                              preferred_element_type=jnp.float32)
                ctx_bf = ctx.astype(jnp.bfloat16)
                ctx_comm[0, rows, cols] = ctx_bf
                out_ref[rows, cols] = ctx_bf
                local_stats[rows, pl.ds(m_lane, 1)] = m
                local_stats[rows, pl.ds(m_lane + 2, 1)] = l
                return 0

            lax.fori_loop(0, R, rbody, 0)

        half = Sq // 2
        top, bot = pl.ds(0, half), pl.ds(half, half)

        NC = 4

        def chunk_hop(c, h):
            specs = (
                (ctx_comm, right, top, pl.ds(c * 2 * DH, 2 * DH)),
                (ctx_comm, left, bot, pl.ds(c * 2 * DH, 2 * DH)),
                (stats_comm, right, pl.ds(c * 4, 4), top),
                (stats_comm, left, pl.ds(c * 4, 4), bot),
            )
            return [pltpu.make_async_remote_copy(
                src_ref=buf.at[h, rows, cols],
                dst_ref=buf.at[h + 1, rows, cols],
                send_sem=send_sems.at[h, c * 4 + f],
                recv_sem=recv_sems.at[h, c * 4 + f],
                device_id=(nbr,),
                device_id_type=pl.DeviceIdType.MESH,
            ) for f, (buf, nbr, rows, cols) in enumerate(specs)]

        hops = [[chunk_hop(c, h) for c in range(NC)]
                for h in range(N_DEV - 1)]

        def pack_stats(c):
            stats_comm[0, pl.ds(c * 4, 4), :] = jnp.transpose(
                local_stats[:, c * 4:c * 4 + 4])

        for c in range(NC):
            attn_head(2 * c)
            attn_head(2 * c + 1)
            pack_stats(c)
            for fl in hops[0][c]:
                fl.start()

        acc = [
            [stats_comm[0, c * 4:c * 4 + 2, :],
             stats_comm[0, c * 4 + 2:c * 4 + 4, :]]
            for c in range(NC)
        ]

        def combine(c, slot, final):
            m_acc, l_acc = acc[c]
            sr = c * 4
            m_in = stats_comm[slot, sr:sr + 2, :]
            l_in = stats_comm[slot, sr + 2:sr + 4, :]
            m_new = jnp.maximum(m_acc, m_in)
            c_old = jnp.exp(m_acc - m_new)
            c_in = jnp.exp(m_in - m_new)
            l_new = l_acc * c_old + l_in * c_in
            acc[c] = [m_new, l_new]
            if final:
                c_old = c_old / l_new
                c_in = c_in / l_new
            c_old = jnp.transpose(c_old)
            c_in = jnp.transpose(c_in)
            for k in range(2):
                cols = pl.ds((c * 2 + k) * DH, DH)
                out_ref[:, cols] = (
                    out_ref[:, cols].astype(jnp.float32) * c_old[:, k:k + 1]
                    + ctx_comm[slot, :, cols].astype(jnp.float32)
                    * c_in[:, k:k + 1]).astype(jnp.bfloat16)

        for h in range(N_DEV - 1):
            last = h == N_DEV - 2
            for c in range(NC):
                for fl in hops[h][c]:
                    fl.wait_recv()
                if not last:
                    for fl in hops[h + 1][c]:
                        fl.start()
                if last:
                    combine(c, h + 1, final=True)
            if not last:
                for c in range(NC):
                    combine(c, h + 1, final=False)
        for hop in hops:
            for chunk in hop:
                for fl in chunk:
                    fl.wait_send()

    ctx2 = pl.pallas_call(
        body,
        out_shape=jax.ShapeDtypeStruct((Sq, HQ * DH), jnp.bfloat16),
        in_specs=[pl.BlockSpec(memory_space=pltpu.VMEM)] * 3,
        out_specs=pl.BlockSpec(memory_space=pltpu.VMEM),
        scratch_shapes=[
            pltpu.VMEM((N_DEV, Sq, HQ * DH), jnp.bfloat16),
            pltpu.VMEM((N_DEV, 16, Sq), jnp.float32),
            pltpu.VMEM((Sq, 16), jnp.float32),
            pltpu.SemaphoreType.DMA((N_DEV - 1, 16)),
            pltpu.SemaphoreType.DMA((N_DEV - 1, 16)),
        ],
        compiler_params=pltpu.CompilerParams(
            collective_id=0, vmem_limit_bytes=46 * 1024 * 1024),
    )(qr, kr, vr)

    ctx_out = ctx2.reshape(R, ngroups, BLK, HQ * DH).transpose(1, 0, 2, 3)
    out = jnp.dot(ctx_out.reshape(Sq, HQ * DH), Wo.astype(jnp.bfloat16),
                  preferred_element_type=jnp.float32)
    return out[None]
